# baseline (speedup 1.0000x reference)
import jax
import jax.numpy as jnp
from jax import lax
from jax.experimental import pallas as pl
from jax.experimental.pallas import tpu as pltpu
from jax.experimental.pallas import tpu_sc as plsc

DIM = 64
NC, NS = 2, 16
NW = NC * NS
CHUNK = 128
GB = 2
TB = 512
WIDE = 2 * DIM


def _pack_body(in_ref, o_ref):
    t = in_ref[...]
    left = jnp.transpose(t[:, :TB])
    right = jnp.transpose(t[:, TB:])
    o_ref[...] = jnp.concatenate([left, right], axis=1)


def _pack_table(wt, vocab):
    nsb = (vocab + 2 * TB - 1) // (2 * TB)
    return pl.pallas_call(
        _pack_body,
        grid=(nsb,),
        in_specs=[pl.BlockSpec((DIM, 2 * TB), lambda c: (0, c))],
        out_specs=pl.BlockSpec((TB, WIDE), lambda c: (c, 0)),
        out_shape=jax.ShapeDtypeStruct((nsb * TB, WIDE), jnp.float32),
    )(wt)


def _gather_body(bpw, NB):
    ngroups = bpw // GB
    npairs = ngroups // 2

    def body(x_hbm, w_hbm, out_hbm, idx_v, q_a, q_b, raw_a, raw_b, t_a, t_b,
             gsem_a, gsem_b, wsem_a, wsem_b):
        wid = lax.axis_index("s") * NC + lax.axis_index("c")
        b0 = wid * bpw
        pltpu.sync_copy(x_hbm.at[pl.ds(b0, bpw)], idx_v)

        iota = lax.iota(jnp.int32, 16)

        def make_qidx(qbuf, g):
            for j in range(GB):
                for kk in range(8):
                    r = idx_v[g * GB + j, pl.ds(kk * 16, 16)]
                    q = ((r >> 10) << 9) | (r & 511)
                    qbuf[j, pl.ds(kk * 16, 16)] = q

        def fire(qbuf, raw, gsem):
            for j in range(GB):
                pltpu.async_copy(
                    w_hbm.at[qbuf.at[j]],
                    raw.at[pl.ds(j * CHUNK, CHUNK)], gsem)

        def drain_gather(raw, gsem):
            pltpu.make_async_copy(
                w_hbm.at[pl.ds(0, GB * CHUNK)], raw, gsem).wait()

        def transpose(raw, t, g):
            BATCH = 16

            def step(tt, carry):
                row_idx = iota + tt * 16
                j = tt // 8
                off = (tt % 8) * 16
                r = idx_v[g * GB + j, pl.ds((tt % 8) * 16, 16)]
                halfoff = ((r >> 9) & 1) << 6
                prev = None
                for k in range(DIM // BATCH):
                    cur = [
                        (d, plsc.load_gather(raw, [row_idx, halfoff + d]))
                        for d in range(k * BATCH, (k + 1) * BATCH)
                    ]
                    if prev is not None:
                        for d, vec in prev:
                            t[d // 8, j, pl.ds((d % 8) * CHUNK + off, 16)] = vec
                    prev = cur
                for d, vec in prev:
                    t[d // 8, j, pl.ds((d % 8) * CHUNK + off, 16)] = vec
                return carry
            lax.fori_loop(0, 16, step, 0)

        def start_writes(t, wsem, g):
            blk = b0 + g * GB
            h = blk // NB
            tc = blk % NB
            for tr in range(8):
                pltpu.async_copy(
                    t.at[tr], out_hbm.at[pl.ds(h * 1024 + tr * NB + tc, GB)],
                    wsem)

        def wait_writes(t, wsem):
            for tr in range(8):
                pltpu.make_async_copy(
                    t.at[tr], out_hbm.at[pl.ds(0, GB)], wsem).wait()

        make_qidx(q_a, 0)
        fire(q_a, raw_a, gsem_a)

        def pair(p, carry):
            ga = 2 * p
            gb = 2 * p + 1
            drain_gather(raw_a, gsem_a)
            make_qidx(q_b, gb)
            fire(q_b, raw_b, gsem_b)

            @pl.when(p > 0)
            def _():
                wait_writes(t_a, wsem_a)

            transpose(raw_a, t_a, ga)
            start_writes(t_a, wsem_a, ga)

            drain_gather(raw_b, gsem_b)

            @pl.when(p < npairs - 1)
            def _():
                make_qidx(q_a, ga + 2)
                fire(q_a, raw_a, gsem_a)

            @pl.when(p > 0)
            def _():
                wait_writes(t_b, wsem_b)

            transpose(raw_b, t_b, gb)
            start_writes(t_b, wsem_b, gb)
            return carry

        lax.fori_loop(0, npairs, pair, 0)
        wait_writes(t_a, wsem_a)
        wait_writes(t_b, wsem_b)
    return body


def kernel(x, weight):
    B, H = x.shape
    V, D = weight.shape
    assert D == DIM and B % CHUNK == 0
    NB = B // CHUNK
    nblocks = H * NB
    assert nblocks % (NW * 2 * GB) == 0
    bpw = nblocks // NW
    xf = x.T.reshape(nblocks, CHUNK).astype(jnp.int32)
    w2 = _pack_table(weight.T, V)
    mesh = plsc.VectorSubcoreMesh(
        core_axis_name="c", subcore_axis_name="s", num_cores=NC, num_subcores=NS
    )
    out = pl.kernel(
        _gather_body(bpw, NB),
        out_type=jax.ShapeDtypeStruct((nblocks * 8, 1024), jnp.float32),
        mesh=mesh,
        compiler_params=pltpu.CompilerParams(
            use_tc_tiling_on_sc=False, needs_layout_passes=False),
        scratch_types=[
            pltpu.VMEM((bpw, CHUNK), jnp.int32),
            pltpu.VMEM((GB, CHUNK), jnp.int32),
            pltpu.VMEM((GB, CHUNK), jnp.int32),
            pltpu.VMEM((GB * CHUNK, WIDE), jnp.float32),
            pltpu.VMEM((GB * CHUNK, WIDE), jnp.float32),
            pltpu.VMEM((8, GB, 1024), jnp.float32),
            pltpu.VMEM((8, GB, 1024), jnp.float32),
            pltpu.SemaphoreType.DMA,
            pltpu.SemaphoreType.DMA,
            pltpu.SemaphoreType.DMA,
            pltpu.SemaphoreType.DMA,
        ],
    )(xf, w2)
    return (out.reshape(H, 8, NB, 8, CHUNK)
               .transpose(2, 4, 0, 1, 3)
               .reshape(B, H, DIM))


# speedup vs baseline: 1.6055x; 1.6055x over previous
import jax
import jax.numpy as jnp
from jax import lax
from jax.experimental import pallas as pl
from jax.experimental.pallas import tpu as pltpu
from jax.experimental.pallas import tpu_sc as plsc

DIM = 64
NC, NS = 2, 16
NW = NC * NS
CHUNK = 128
GB = 2
PAD = CHUNK + 1


def _gather_body(bpw, NB):
    ngroups = bpw // GB
    npairs = ngroups // 2

    def body(x_hbm, w_hbm, out_hbm, idx_v, raw_a, raw_b, t_a, t_b,
             gsem_a, gsem_b, wsem_a, wsem_b):
        wid = lax.axis_index("s") * NC + lax.axis_index("c")
        b0 = wid * bpw
        pltpu.sync_copy(x_hbm.at[pl.ds(b0, bpw)], idx_v)

        def fire(raw, gsem, g):
            for j in range(GB):
                pltpu.async_copy(
                    w_hbm.at[idx_v.at[g * GB + j]],
                    raw.at[pl.ds(j * CHUNK, CHUNK)], gsem)

        def drain_gather(raw, gsem):
            pltpu.make_async_copy(
                w_hbm.at[pl.ds(0, GB * CHUNK)], raw, gsem).wait()

        iota = lax.iota(jnp.int32, 16)
        patterns = [
            ((iota + dd) >> 3) * (GB * 8) + ((iota + dd) & 7)
            for dd in range(0, DIM, 16)
        ]

        def transpose(raw, t):
            RB = 4

            def batch(q):
                out = []
                for rr in range(RB):
                    r = q * RB + rr
                    j8 = (r // CHUNK) * 8
                    l = r % CHUNK
                    for k in range(DIM // 16):
                        vec = raw[r, pl.ds(k * 16, 16)]
                        out.append((patterns[k] + j8, jnp.full((16,), l,
                                                               jnp.int32), vec))
                return out

            def flush(items):
                for rows, lanes, vec in items:
                    plsc.store_scatter(t, [rows, lanes], vec)

            def step(q, carry):
                prev = batch(2 * q)
                cur = batch(2 * q + 1)
                flush(prev)
                flush(cur)
                return carry
            lax.fori_loop(0, GB * CHUNK // (2 * RB), step, 0)

        def start_writes(t, wsem, g):
            blk = b0 + g * GB
            h = blk // NB
            tc = blk % NB
            for tr in range(8):
                pltpu.async_copy(
                    t.at[pl.ds(tr * GB * 8, GB * 8), pl.ds(0, CHUNK)],
                    out_hbm.at[pl.ds((h * 1024 + tr * NB + tc) * 8, GB * 8)],
                    wsem)

        def wait_writes(t, wsem):
            for tr in range(8):
                pltpu.make_async_copy(
                    t.at[pl.ds(tr * GB * 8, GB * 8), pl.ds(0, CHUNK)],
                    out_hbm.at[pl.ds(0, GB * 8)], wsem).wait()

        fire(raw_a, gsem_a, 0)

        def pair(p, carry):
            ga = 2 * p
            gb = 2 * p + 1
            drain_gather(raw_a, gsem_a)
            fire(raw_b, gsem_b, gb)

            @pl.when(p > 0)
            def _():
                wait_writes(t_a, wsem_a)

            transpose(raw_a, t_a)
            start_writes(t_a, wsem_a, ga)

            drain_gather(raw_b, gsem_b)

            @pl.when(p < npairs - 1)
            def _():
                fire(raw_a, gsem_a, ga + 2)

            @pl.when(p > 0)
            def _():
                wait_writes(t_b, wsem_b)

            transpose(raw_b, t_b)
            start_writes(t_b, wsem_b, gb)
            return carry

        lax.fori_loop(0, npairs, pair, 0)
        wait_writes(t_a, wsem_a)
        wait_writes(t_b, wsem_b)
    return body


def kernel(x, weight):
    B, H = x.shape
    V, D = weight.shape
    assert D == DIM and B % CHUNK == 0
    NB = B // CHUNK
    nblocks = H * NB
    assert nblocks % (NW * 2 * GB) == 0
    bpw = nblocks // NW
    xf = x.T.reshape(nblocks, CHUNK).astype(jnp.int32)
    mesh = plsc.VectorSubcoreMesh(
        core_axis_name="c", subcore_axis_name="s", num_cores=NC, num_subcores=NS
    )
    out = pl.kernel(
        _gather_body(bpw, NB),
        out_type=jax.ShapeDtypeStruct((nblocks * 64, CHUNK), jnp.float32),
        mesh=mesh,
        compiler_params=pltpu.CompilerParams(
            use_tc_tiling_on_sc=False, needs_layout_passes=False),
        scratch_types=[
            pltpu.VMEM((bpw, CHUNK), jnp.int32),
            pltpu.VMEM((GB * CHUNK, DIM), jnp.float32),
            pltpu.VMEM((GB * CHUNK, DIM), jnp.float32),
            pltpu.VMEM((8 * GB * 8, PAD), jnp.float32),
            pltpu.VMEM((8 * GB * 8, PAD), jnp.float32),
            pltpu.SemaphoreType.DMA,
            pltpu.SemaphoreType.DMA,
            pltpu.SemaphoreType.DMA,
            pltpu.SemaphoreType.DMA,
        ],
    )(xf, weight)
    return (out.reshape(H, 8, NB, 8, CHUNK)
               .transpose(2, 4, 0, 1, 3)
               .reshape(B, H, DIM))


# speedup vs baseline: 1.6060x; 1.0003x over previous
import jax
import jax.numpy as jnp
from jax import lax
from jax.experimental import pallas as pl
from jax.experimental.pallas import tpu as pltpu
from jax.experimental.pallas import tpu_sc as plsc

DIM = 64
NC, NS = 2, 16
NW = NC * NS
CHUNK = 128
GB = 2
PAD = CHUNK + 1


def _gather_body(bpw, NB):
    ngroups = bpw // GB
    npairs = ngroups // 2

    def body(x_hbm, w_hbm, out_hbm, idx_v, raw_a, raw_b, t_a, t_b,
             gsem_a, gsem_b, wsem_a, wsem_b):
        wid = lax.axis_index("s") * NC + lax.axis_index("c")
        b0 = wid * bpw
        pltpu.sync_copy(x_hbm.at[pl.ds(b0, bpw)], idx_v)

        def fire(raw, gsem, g):
            for j in range(GB):
                pltpu.async_copy(
                    w_hbm.at[idx_v.at[g * GB + j]],
                    raw.at[pl.ds(j * CHUNK, CHUNK)], gsem)

        def drain_gather(raw, gsem):
            pltpu.make_async_copy(
                w_hbm.at[pl.ds(0, GB * CHUNK)], raw, gsem).wait()

        iota = lax.iota(jnp.int32, 16)
        patterns = [
            ((iota + dd) >> 3) * (GB * 8) + ((iota + dd) & 7)
            for dd in range(0, DIM, 16)
        ]

        def transpose(raw, t):
            RB = 4

            def batch(q):
                out = []
                for rr in range(RB):
                    r = q * RB + rr
                    j8 = (r // CHUNK) * 8
                    l = r % CHUNK
                    for k in range(DIM // 16):
                        vec = raw[r, pl.ds(k * 16, 16)]
                        out.append((patterns[k] + j8, jnp.full((16,), l,
                                                               jnp.int32), vec))
                return out

            def flush(items):
                for rows, lanes, vec in items:
                    plsc.store_scatter(t, [rows, lanes], vec)

            def step(q, carry):
                prev = batch(2 * q)
                cur = batch(2 * q + 1)
                flush(prev)
                flush(cur)
                return carry
            lax.fori_loop(0, GB * CHUNK // (2 * RB), step, 0)

        def start_writes(t, wsem, g):
            blk = b0 + g * GB
            h = blk // NB
            tc = blk % NB
            for tr in range(8):
                pltpu.async_copy(
                    t.at[pl.ds(tr * GB * 8, GB * 8), pl.ds(0, CHUNK)],
                    out_hbm.at[pl.ds((h * 1024 + tr * NB + tc) * 8, GB * 8)],
                    wsem)

        def wait_writes(t, wsem):
            for tr in range(8):
                pltpu.make_async_copy(
                    t.at[pl.ds(tr * GB * 8, GB * 8), pl.ds(0, CHUNK)],
                    out_hbm.at[pl.ds(0, GB * 8)], wsem).wait()

        fire(raw_a, gsem_a, 0)

        def pair(p, carry):
            ga = 2 * p
            gb = 2 * p + 1
            drain_gather(raw_a, gsem_a)
            fire(raw_b, gsem_b, gb)

            @pl.when(p > 0)
            def _():
                wait_writes(t_a, wsem_a)

            transpose(raw_a, t_a)

            @pl.when(p < npairs - 1)
            def _():
                fire(raw_a, gsem_a, ga + 2)

            start_writes(t_a, wsem_a, ga)

            drain_gather(raw_b, gsem_b)

            @pl.when(p > 0)
            def _():
                wait_writes(t_b, wsem_b)

            transpose(raw_b, t_b)
            start_writes(t_b, wsem_b, gb)
            return carry

        lax.fori_loop(0, npairs, pair, 0)
        wait_writes(t_a, wsem_a)
        wait_writes(t_b, wsem_b)
    return body


def kernel(x, weight):
    B, H = x.shape
    V, D = weight.shape
    assert D == DIM and B % CHUNK == 0
    NB = B // CHUNK
    nblocks = H * NB
    assert nblocks % (NW * 2 * GB) == 0
    bpw = nblocks // NW
    xf = x.T.reshape(nblocks, CHUNK).astype(jnp.int32)
    mesh = plsc.VectorSubcoreMesh(
        core_axis_name="c", subcore_axis_name="s", num_cores=NC, num_subcores=NS
    )
    out = pl.kernel(
        _gather_body(bpw, NB),
        out_type=jax.ShapeDtypeStruct((nblocks * 64, CHUNK), jnp.float32),
        mesh=mesh,
        compiler_params=pltpu.CompilerParams(
            use_tc_tiling_on_sc=False, needs_layout_passes=False),
        scratch_types=[
            pltpu.VMEM((bpw, CHUNK), jnp.int32),
            pltpu.VMEM((GB * CHUNK, DIM), jnp.float32),
            pltpu.VMEM((GB * CHUNK, DIM), jnp.float32),
            pltpu.VMEM((8 * GB * 8, PAD), jnp.float32),
            pltpu.VMEM((8 * GB * 8, PAD), jnp.float32),
            pltpu.SemaphoreType.DMA,
            pltpu.SemaphoreType.DMA,
            pltpu.SemaphoreType.DMA,
            pltpu.SemaphoreType.DMA,
        ],
    )(xf, weight)
    return (out.reshape(H, 8, NB, 8, CHUNK)
               .transpose(2, 4, 0, 1, 3)
               .reshape(B, H, DIM))


# speedup vs baseline: 1.6932x; 1.0543x over previous
import jax
import jax.numpy as jnp
from jax import lax
from jax.experimental import pallas as pl
from jax.experimental.pallas import tpu as pltpu
from jax.experimental.pallas import tpu_sc as plsc

DIM = 64
NC, NS = 2, 16
NW = NC * NS
CHUNK = 128
GB = 2
PAD = CHUNK + 1


def _gather_body(bpw, NB):
    ngroups = bpw // GB
    npairs = ngroups // 2

    def body(x_hbm, w_hbm, out_hbm, idx_v, raw_a, raw_b, t_a, t_b,
             gsem_a, gsem_b, wsem_a, wsem_b):
        wid = lax.axis_index("s") * NC + lax.axis_index("c")
        b0 = wid * bpw
        pltpu.sync_copy(x_hbm.at[pl.ds(b0, bpw)], idx_v)

        def fire(raw, gsem, g):
            for j in range(GB):
                pltpu.async_copy(
                    w_hbm.at[idx_v.at[g * GB + j]],
                    raw.at[pl.ds(j * CHUNK, CHUNK)], gsem)

        def drain_gather(raw, gsem):
            pltpu.make_async_copy(
                w_hbm.at[pl.ds(0, GB * CHUNK)], raw, gsem).wait()

        iota = lax.iota(jnp.int32, 16)
        patterns = [
            ((iota + dd) >> 3) * (GB * 8) + ((iota + dd) & 7)
            for dd in range(0, DIM, 16)
        ]

        def transpose(raw, t):
            RB = 4

            def batch(q):
                out = []
                for rr in range(RB):
                    r = q * RB + rr
                    j8 = (r // CHUNK) * 8
                    l = r % CHUNK
                    for k in range(DIM // 16):
                        vec = raw[r, pl.ds(k * 16, 16)]
                        out.append((patterns[k] + j8, jnp.full((16,), l,
                                                               jnp.int32), vec))
                return out

            def flush(items):
                for rows, lanes, vec in items:
                    plsc.store_scatter(t, [rows, lanes], vec)

            def step(q, carry):
                prev = batch(2 * q)
                cur = batch(2 * q + 1)
                flush(prev)
                flush(cur)
                return carry
            lax.fori_loop(0, GB * CHUNK // (2 * RB), step, 0)

        def start_writes(t, wsem, g):
            blk = b0 + g * GB
            h = blk // NB
            tc = blk % NB
            for tr in range(8):
                pltpu.async_copy(
                    t.at[pl.ds(tr * GB * 8, GB * 8), pl.ds(0, CHUNK)],
                    out_hbm.at[pl.ds((h * 1024 + tr * NB + tc) * 8, GB * 8)],
                    wsem)

        def wait_writes(t, wsem):
            for tr in range(8):
                pltpu.make_async_copy(
                    t.at[pl.ds(tr * GB * 8, GB * 8), pl.ds(0, CHUNK)],
                    out_hbm.at[pl.ds(0, GB * 8)], wsem).wait()

        fire(raw_a, gsem_a, 0)

        def pair(p, carry):
            ga = 2 * p
            gb = 2 * p + 1
            drain_gather(raw_a, gsem_a)
            fire(raw_b, gsem_b, gb)

            @pl.when(p > 0)
            def _():
                wait_writes(t_a, wsem_a)

            transpose(raw_a, t_a)

            @pl.when(p < npairs - 1)
            def _():
                fire(raw_a, gsem_a, ga + 2)

            start_writes(t_a, wsem_a, ga)

            drain_gather(raw_b, gsem_b)

            @pl.when(p > 0)
            def _():
                wait_writes(t_b, wsem_b)

            transpose(raw_b, t_b)
            start_writes(t_b, wsem_b, gb)
            return carry

        lax.fori_loop(0, npairs, pair, 0)
        wait_writes(t_a, wsem_a)
        wait_writes(t_b, wsem_b)
    return body


def kernel(x, weight):
    B, H = x.shape
    V, D = weight.shape
    assert D == DIM and B % CHUNK == 0
    NB = B // CHUNK
    nblocks = H * NB
    assert nblocks % (NW * 2 * GB) == 0
    bpw = nblocks // NW
    xf = x.T.reshape(nblocks, CHUNK).astype(jnp.int32)
    wp = jnp.pad(weight, ((0, 0), (0, DIM)))
    mesh = plsc.VectorSubcoreMesh(
        core_axis_name="c", subcore_axis_name="s", num_cores=NC, num_subcores=NS
    )
    out = pl.kernel(
        _gather_body(bpw, NB),
        out_type=jax.ShapeDtypeStruct((nblocks * 64, CHUNK), jnp.float32),
        mesh=mesh,
        compiler_params=pltpu.CompilerParams(
            use_tc_tiling_on_sc=False, needs_layout_passes=False),
        scratch_types=[
            pltpu.VMEM((bpw, CHUNK), jnp.int32),
            pltpu.VMEM((GB * CHUNK, 2 * DIM), jnp.float32),
            pltpu.VMEM((GB * CHUNK, 2 * DIM), jnp.float32),
            pltpu.VMEM((8 * GB * 8, PAD), jnp.float32),
            pltpu.VMEM((8 * GB * 8, PAD), jnp.float32),
            pltpu.SemaphoreType.DMA,
            pltpu.SemaphoreType.DMA,
            pltpu.SemaphoreType.DMA,
            pltpu.SemaphoreType.DMA,
        ],
    )(xf, wp)
    return (out.reshape(H, 8, NB, 8, CHUNK)
               .transpose(2, 4, 0, 1, 3)
               .reshape(B, H, DIM))
